# Initial kernel scaffold; baseline (speedup 1.0000x reference)
import functools

import jax
import jax.numpy as jnp
from jax import lax
from jax.experimental import pallas as pl
from jax.experimental.pallas import tpu as pltpu
from jax.experimental.pallas import tpu_sc as plsc

N_NODES = 10000
D = 128
E = 320000

K = 125
NB = E // K
NW = 32
BPW = NB // NW
ROWS_PER_TILE = N_NODES // 16
LANES = 16


def _sc_spmm(x, row2d, col2d, w2d):
    mesh = plsc.VectorSubcoreMesh(core_axis_name="c", subcore_axis_name="s")

    @functools.partial(
        pl.kernel,
        out_type=jax.ShapeDtypeStruct((2, N_NODES, D), jnp.float32),
        mesh=mesh,
        scratch_types=dict(
            row_v=pltpu.MemoryRef((BPW, K), jnp.int32, memory_space=pltpu.VMEM),
            col_v=pltpu.MemoryRef((BPW, K), jnp.int32, memory_space=pltpu.VMEM),
            w_v=pltpu.MemoryRef((BPW, K), jnp.float32, memory_space=pltpu.VMEM),
            rows_v=pltpu.MemoryRef((K, D), jnp.float32, memory_space=pltpu.VMEM),
            zero_v=pltpu.MemoryRef((K, D), jnp.float32, memory_space=pltpu.VMEM),
            accum=pltpu.MemoryRef((N_NODES, D), jnp.float32,
                                  memory_space=pltpu.VMEM_SHARED),
            sem=pltpu.SemaphoreType.DMA,
        ),
    )
    def k(x_hbm, row_hbm, col_hbm, w_hbm, out_hbm,
          row_v, col_v, w_v, rows_v, zero_v, accum, sem):
        cid = lax.axis_index("c")
        sid = lax.axis_index("s")
        wid = cid * 16 + sid

        def _z(i, _):
            for c in range(D // LANES):
                zero_v[i, pl.ds(c * LANES, LANES)] = jnp.zeros(
                    (LANES,), jnp.float32)
            return 0
        lax.fori_loop(0, K, _z, 0)
        for r in range(ROWS_PER_TILE // K):
            pltpu.sync_copy(zero_v,
                            accum.at[pl.ds(sid * ROWS_PER_TILE + r * K, K)])

        pltpu.sync_copy(row_hbm.at[pl.ds(wid * BPW, BPW)], row_v)
        pltpu.sync_copy(col_hbm.at[pl.ds(wid * BPW, BPW)], col_v)
        pltpu.sync_copy(w_hbm.at[pl.ds(wid * BPW, BPW)], w_v)

        plsc.subcore_barrier()

        def body(b, _):
            pltpu.async_copy(x_hbm.at[col_v.at[b]], rows_v, sem).wait()

            def scale(e, _):
                bi = jnp.full((LANES,), b, jnp.int32)
                ei = jnp.full((LANES,), e, jnp.int32)
                wv = plsc.load_gather(w_v, [bi, ei])
                for c in range(D // LANES):
                    sl = pl.ds(c * LANES, LANES)
                    rows_v[e, sl] = rows_v[e, sl] * wv
                return 0
            lax.fori_loop(0, K, scale, 0)

            pltpu.sync_copy(rows_v, accum.at[row_v.at[b]], add=True)
            return 0

        lax.fori_loop(0, BPW, body, 0)

        plsc.subcore_barrier()

        pltpu.sync_copy(
            accum.at[pl.ds(sid * ROWS_PER_TILE, ROWS_PER_TILE)],
            out_hbm.at[cid, pl.ds(sid * ROWS_PER_TILE, ROWS_PER_TILE)])

    return k(x, row2d, col2d, w2d)


def _add_body(a_ref, b_ref, o_ref):
    o_ref[...] = a_ref[...] + b_ref[...]


def _combine(partials):
    grid = 8
    blk = N_NODES // grid
    return pl.pallas_call(
        _add_body,
        grid=(grid,),
        in_specs=[pl.BlockSpec((blk, D), lambda i: (i, 0)),
                  pl.BlockSpec((blk, D), lambda i: (i, 0))],
        out_specs=pl.BlockSpec((blk, D), lambda i: (i, 0)),
        out_shape=jax.ShapeDtypeStruct((N_NODES, D), jnp.float32),
    )(partials[0], partials[1])


@jax.jit
def kernel(x, edge_index, edge_weight):
    row2d = edge_index[0].reshape(NB, K)
    col2d = edge_index[1].reshape(NB, K)
    w2d = edge_weight.astype(jnp.float32).reshape(NB, K)
    partials = _sc_spmm(x, row2d, col2d, w2d)
    return _combine(partials)


# speedup vs baseline: 7.5461x; 7.5461x over previous
import functools

import jax
import jax.numpy as jnp
from jax import lax
from jax.experimental import pallas as pl
from jax.experimental.pallas import tpu as pltpu
from jax.experimental.pallas import tpu_sc as plsc

N_NODES = 10000
D = 128
E = 320000

NW = 32
K = 40
EPW = E // NW
BPW = EPW // K
NPAIR = BPW // 2
STRIPE = 1000
ZCHUNK = 40
LANES = 16
UNROLL = 4


def _sc_spmm(x, row2, col2, w2):
    mesh = plsc.VectorSubcoreMesh(core_axis_name="c", subcore_axis_name="s")

    @functools.partial(
        pl.kernel,
        out_type=jax.ShapeDtypeStruct((2, N_NODES, D), jnp.float32),
        mesh=mesh,
        compiler_params=pltpu.CompilerParams(needs_layout_passes=False),
        scratch_types=dict(
            row_v=pltpu.VMEM((EPW,), jnp.int32),
            col_v=pltpu.VMEM((EPW,), jnp.int32),
            w_v=pltpu.VMEM((EPW,), jnp.float32),
            rows0=pltpu.VMEM((K, D), jnp.float32),
            rows1=pltpu.VMEM((K, D), jnp.float32),
            g0=pltpu.SemaphoreType.DMA,
            g1=pltpu.SemaphoreType.DMA,
            s0=pltpu.SemaphoreType.DMA,
            s1=pltpu.SemaphoreType.DMA,
            accum=pltpu.VMEM_SHARED((N_NODES, D), jnp.float32),
        ),
    )
    def k(x_hbm, row_hbm, col_hbm, w_hbm, out_hbm,
          row_v, col_v, w_v, rows0, rows1, g0, g1, s0, s1, accum):
        cid = lax.axis_index("c")
        sid = lax.axis_index("s")
        wid = cid * 16 + sid

        def _z(i, _):
            for c in range(D // LANES):
                rows0[i, pl.ds(c * LANES, LANES)] = jnp.zeros(
                    (LANES,), jnp.float32)
            return 0
        lax.fori_loop(0, K, _z, 0)

        @pl.when(sid < N_NODES // STRIPE)
        def _zero_stripe():
            for r in range(STRIPE // ZCHUNK):
                pltpu.sync_copy(
                    rows0.at[pl.ds(0, ZCHUNK)],
                    accum.at[pl.ds(sid * STRIPE + r * ZCHUNK, ZCHUNK)])

        pltpu.sync_copy(row_hbm.at[wid], row_v)
        pltpu.sync_copy(col_hbm.at[wid], col_v)
        pltpu.sync_copy(w_hbm.at[wid], w_v)

        plsc.subcore_barrier()

        def gather(b, buf, sem):
            pltpu.async_copy(x_hbm.at[col_v.at[pl.ds(b * K, K)]], buf, sem)

        def gather_wait(buf, sem):
            pltpu.make_async_copy(
                x_hbm.at[col_v.at[pl.ds(0, K)]], buf, sem).wait()

        def scatter(b, buf, sem):
            pltpu.async_copy(
                buf, accum.at[row_v.at[pl.ds(b * K, K)]], sem, add=True)

        def scatter_wait(buf, sem):
            pltpu.make_async_copy(
                buf, accum.at[row_v.at[pl.ds(0, K)]], sem).wait()

        def scale(b, buf):
            base = b * K
            def step(j, _):
                for u in range(UNROLL):
                    e = j * UNROLL + u
                    fi = jnp.full((LANES,), base + e, jnp.int32)
                    wv = plsc.load_gather(w_v, [fi])
                    for c in range(D // LANES):
                        sl = pl.ds(c * LANES, LANES)
                        buf[e, sl] = buf[e, sl] * wv
                return 0
            lax.fori_loop(0, K // UNROLL, step, 0)

        gather(0, rows0, g0)
        gather(1, rows1, g1)

        def pair(t, _):
            b = 2 * t
            gather_wait(rows0, g0)
            scale(b, rows0)
            scatter(b, rows0, s0)
            gather_wait(rows1, g1)
            scale(b + 1, rows1)
            scatter(b + 1, rows1, s1)

            @pl.when(t < NPAIR - 1)
            def _prefetch():
                scatter_wait(rows0, s0)
                gather(b + 2, rows0, g0)
                scatter_wait(rows1, s1)
                gather(b + 3, rows1, g1)
            return 0

        lax.fori_loop(0, NPAIR, pair, 0)

        scatter_wait(rows0, s0)
        scatter_wait(rows1, s1)

        plsc.subcore_barrier()

        @pl.when(sid < N_NODES // STRIPE)
        def _writeback():
            pltpu.sync_copy(
                accum.at[pl.ds(sid * STRIPE, STRIPE)],
                out_hbm.at[cid, pl.ds(sid * STRIPE, STRIPE)])

    return k(x, row2, col2, w2)


def _add_body(a_ref, b_ref, o_ref):
    o_ref[...] = a_ref[...] + b_ref[...]


def _combine(partials):
    grid = 10
    blk = N_NODES // grid
    return pl.pallas_call(
        _add_body,
        grid=(grid,),
        in_specs=[pl.BlockSpec((blk, D), lambda i: (i, 0)),
                  pl.BlockSpec((blk, D), lambda i: (i, 0))],
        out_specs=pl.BlockSpec((blk, D), lambda i: (i, 0)),
        out_shape=jax.ShapeDtypeStruct((N_NODES, D), jnp.float32),
    )(partials[0], partials[1])


@jax.jit
def kernel(x, edge_index, edge_weight):
    row2 = edge_index[0].reshape(NW, EPW)
    col2 = edge_index[1].reshape(NW, EPW)
    w2 = edge_weight.astype(jnp.float32).reshape(NW, EPW)
    partials = _sc_spmm(x, row2, col2, w2)
    return _combine(partials)


# speedup vs baseline: 9.0175x; 1.1950x over previous
import functools

import jax
import jax.numpy as jnp
from jax import lax
from jax.experimental import pallas as pl
from jax.experimental.pallas import tpu as pltpu
from jax.experimental.pallas import tpu_sc as plsc

N_NODES = 10000
D = 128
E = 320000

NW = 32
K = 40
EPW = E // NW
BPW = EPW // K
NBUF = 4
LEAD = 2
STRIPE = 1000
ZCHUNK = 40
LANES = 16
UNROLL = 4


def _sc_spmm(x, row2, col2, w2):
    mesh = plsc.VectorSubcoreMesh(core_axis_name="c", subcore_axis_name="s")

    @functools.partial(
        pl.kernel,
        out_type=jax.ShapeDtypeStruct((2, N_NODES, D), jnp.float32),
        mesh=mesh,
        compiler_params=pltpu.CompilerParams(needs_layout_passes=False),
        scratch_types=dict(
            row_v=pltpu.VMEM((EPW,), jnp.int32),
            col_v=pltpu.VMEM((EPW,), jnp.int32),
            w_v=pltpu.VMEM((EPW,), jnp.float32),
            rows=[pltpu.VMEM((K, D), jnp.float32) for _ in range(NBUF)],
            gsem=[pltpu.SemaphoreType.DMA for _ in range(NBUF)],
            ssem=[pltpu.SemaphoreType.DMA for _ in range(NBUF)],
            accum=pltpu.VMEM_SHARED((N_NODES, D), jnp.float32),
        ),
    )
    def k(x_hbm, row_hbm, col_hbm, w_hbm, out_hbm,
          row_v, col_v, w_v, rows, gsem, ssem, accum):
        cid = lax.axis_index("c")
        sid = lax.axis_index("s")
        wid = cid * 16 + sid

        def _z(i, _):
            for c in range(D // LANES):
                rows[0][i, pl.ds(c * LANES, LANES)] = jnp.zeros(
                    (LANES,), jnp.float32)
            return 0
        lax.fori_loop(0, K, _z, 0)

        @pl.when(sid < N_NODES // STRIPE)
        def _zero_stripe():
            for r in range(STRIPE // ZCHUNK):
                pltpu.sync_copy(
                    rows[0].at[pl.ds(0, ZCHUNK)],
                    accum.at[pl.ds(sid * STRIPE + r * ZCHUNK, ZCHUNK)])

        pltpu.sync_copy(row_hbm.at[wid], row_v)
        pltpu.sync_copy(col_hbm.at[wid], col_v)
        pltpu.sync_copy(w_hbm.at[wid], w_v)

        plsc.subcore_barrier()

        def gather(b, buf, sem):
            pltpu.async_copy(x_hbm.at[col_v.at[pl.ds(b * K, K)]], buf, sem)

        def gather_wait(buf, sem):
            pltpu.make_async_copy(
                x_hbm.at[col_v.at[pl.ds(0, K)]], buf, sem).wait()

        def scatter(b, buf, sem):
            pltpu.async_copy(
                buf, accum.at[row_v.at[pl.ds(b * K, K)]], sem, add=True)

        def scatter_wait(buf, sem):
            pltpu.make_async_copy(
                buf, accum.at[row_v.at[pl.ds(0, K)]], sem).wait()

        def scale(b, buf):
            base = b * K
            def step(j, _):
                for u in range(UNROLL):
                    e = j * UNROLL + u
                    fi = jnp.full((LANES,), base + e, jnp.int32)
                    wv = plsc.load_gather(w_v, [fi])
                    for c in range(D // LANES):
                        sl = pl.ds(c * LANES, LANES)
                        buf[e, sl] = buf[e, sl] * wv
                return 0
            lax.fori_loop(0, K // UNROLL, step, 0)

        for u in range(LEAD):
            gather(u, rows[u], gsem[u])

        def ring(t, _):
            for u in range(NBUF):
                b = NBUF * t + u
                gather_wait(rows[u], gsem[u])
                scale(b, rows[u])
                scatter(b, rows[u], ssem[u])
                v = (u + LEAD) % NBUF

                @pl.when(b >= NBUF - LEAD)
                def _drain():
                    scatter_wait(rows[v], ssem[v])

                @pl.when(b + LEAD < BPW)
                def _prefetch():
                    gather(b + LEAD, rows[v], gsem[v])
            return 0

        lax.fori_loop(0, BPW // NBUF, ring, 0)

        for b in range(BPW - BPW % NBUF, BPW):
            u = b % NBUF
            gather_wait(rows[u], gsem[u])
            scale(b, rows[u])
            scatter(b, rows[u], ssem[u])
        for b in range(BPW - NBUF, BPW):
            u = b % NBUF
            scatter_wait(rows[u], ssem[u])

        plsc.subcore_barrier()

        @pl.when(sid < N_NODES // STRIPE)
        def _writeback():
            pltpu.sync_copy(
                accum.at[pl.ds(sid * STRIPE, STRIPE)],
                out_hbm.at[cid, pl.ds(sid * STRIPE, STRIPE)])

    return k(x, row2, col2, w2)


def _add_body(a_ref, b_ref, o_ref):
    o_ref[...] = a_ref[...] + b_ref[...]


def _combine(partials):
    grid = 10
    blk = N_NODES // grid
    return pl.pallas_call(
        _add_body,
        grid=(grid,),
        in_specs=[pl.BlockSpec((blk, D), lambda i: (i, 0)),
                  pl.BlockSpec((blk, D), lambda i: (i, 0))],
        out_specs=pl.BlockSpec((blk, D), lambda i: (i, 0)),
        out_shape=jax.ShapeDtypeStruct((N_NODES, D), jnp.float32),
    )(partials[0], partials[1])


@jax.jit
def kernel(x, edge_index, edge_weight):
    row2 = edge_index[0].reshape(NW, EPW)
    col2 = edge_index[1].reshape(NW, EPW)
    w2 = edge_weight.astype(jnp.float32).reshape(NW, EPW)
    partials = _sc_spmm(x, row2, col2, w2)
    return _combine(partials)
